# baseline (speedup 1.0000x reference)
import functools

import jax
import jax.numpy as jnp
from jax import lax
from jax.experimental import pallas as pl
from jax.experimental.pallas import tpu as pltpu
from jax.experimental.pallas import tpu_sc as plsc

E = 8
TOP_K = 2
TILE_M = 256

NC, NS, LANES = 2, 16, 16
NW = NC * NS
_VMESH = dict(core_axis_name="c", subcore_axis_name="s")


def _sc_compiler_params():
    cp = pltpu.CompilerParams()
    if "needs_layout_passes" in pltpu.CompilerParams.__dataclass_fields__:
        import dataclasses
        cp = dataclasses.replace(cp, needs_layout_passes=False)
    return cp


def _sc_build_table(slot, tokens, pad_n, n_tokens):
    na = slot.shape[0]
    mask = n_tokens - 1

    @functools.partial(
        pl.kernel,
        mesh=plsc.VectorSubcoreMesh(**_VMESH),
        out_type=jax.ShapeDtypeStruct((pad_n,), jnp.int32),
        scratch_types=[
            pltpu.VMEM((na,), jnp.int32),
            pltpu.VMEM((na,), jnp.int32),
            pltpu.VMEM((pad_n,), jnp.int32),
        ],
        compiler_params=_sc_compiler_params(),
    )
    def k(slot_hbm, tok_hbm, out_hbm, slot_v, tok_v, table_v):
        wid = lax.axis_index("s") * NC + lax.axis_index("c")

        @pl.when(wid == 0)
        def _():
            pltpu.sync_copy(slot_hbm, slot_v)
            pltpu.sync_copy(tok_hbm, tok_v)

            @pl.loop(0, pad_n // LANES)
            def _(i):
                base = i * LANES
                table_v[pl.ds(base, LANES)] = (
                    (base + lax.iota(jnp.int32, LANES)) & mask)

            @pl.loop(0, na // LANES)
            def _(i):
                idx = slot_v[pl.ds(i * LANES, LANES)]
                val = tok_v[pl.ds(i * LANES, LANES)]
                plsc.store_scatter(table_v, [idx], val)

            pltpu.sync_copy(table_v, out_hbm)

    return k(slot, tokens)


def _sc_gather_rows(table, idx, chunk=64):
    b = idx.shape[0]
    d = table.shape[1]
    b_per_w = b // NW
    nchunk = b_per_w // chunk
    idx3 = idx.reshape(NW, nchunk, chunk)

    nbuf = min(3, nchunk)

    @functools.partial(
        pl.kernel,
        mesh=plsc.VectorSubcoreMesh(**_VMESH),
        out_type=jax.ShapeDtypeStruct((b, d), table.dtype),
        scratch_types=(
            [pltpu.VMEM((nchunk, chunk), jnp.int32)]
            + [pltpu.VMEM((chunk, d), table.dtype)] * nbuf
            + [pltpu.SemaphoreType.DMA] * (2 * nbuf)
        ),
        compiler_params=_sc_compiler_params(),
    )
    def k(table_hbm, idx_hbm, out_hbm, idx_v, *bufs_and_sems):
        bufs = bufs_and_sems[:nbuf]
        gsem = bufs_and_sems[nbuf:2 * nbuf]
        wsem = bufs_and_sems[2 * nbuf:]
        wid = lax.axis_index("s") * NC + lax.axis_index("c")
        base = wid * b_per_w
        pltpu.sync_copy(idx_hbm.at[wid], idx_v)
        for bb in range(nbuf):
            pltpu.async_copy(table_hbm.at[idx_v.at[bb]], bufs[bb], gsem[bb])

        @pl.loop(0, nchunk)
        def _(c):
            for bb in range(nbuf):
                @pl.when(c % nbuf == bb)
                def _():
                    pltpu.make_async_copy(
                        table_hbm.at[idx_v.at[c]], bufs[bb], gsem[bb]).wait()
                    dst = out_hbm.at[pl.ds(base + c * chunk, chunk)]
                    pltpu.async_copy(bufs[bb], dst, wsem[bb])

                    @pl.when(c + nbuf < nchunk)
                    def _():
                        pltpu.make_async_copy(bufs[bb], dst, wsem[bb]).wait()
                        pltpu.async_copy(
                            table_hbm.at[idx_v.at[c + nbuf]],
                            bufs[bb], gsem[bb])

        for bb in range(nbuf):
            c_last = nchunk - 1 - ((nchunk - 1 - bb) % nbuf)
            pltpu.make_async_copy(
                bufs[bb],
                out_hbm.at[pl.ds(base + c_last * chunk, chunk)],
                wsem[bb]).wait()

    return k(table, idx3)


def _router_body(x_ref, wr_ref, br_ref, idx_ref, gates_ref, zsq_ref):
    t = pl.program_id(0)
    logits = jnp.dot(x_ref[...], wr_ref[...],
                     preferred_element_type=jnp.float32) + br_ref[...]
    rows = logits.shape[0]
    lane = jax.lax.broadcasted_iota(jnp.int32, (rows, E), 1)
    v1 = jnp.max(logits, axis=-1, keepdims=True)
    i1 = jnp.min(jnp.where(logits == v1, lane, E), axis=-1, keepdims=True)
    masked = jnp.where(lane == i1, -jnp.inf, logits)
    v2 = jnp.max(masked, axis=-1, keepdims=True)
    i2 = jnp.min(jnp.where(masked == v2, lane, E), axis=-1, keepdims=True)
    e1 = jnp.exp(v2 - v1)
    denom = 1.0 + e1
    idx_ref[...] = jnp.concatenate([i1, i2], axis=-1)
    gates_ref[...] = jnp.concatenate([1.0 / denom, e1 / denom], axis=-1)
    z = v1 + jnp.log1p(e1)

    @pl.when(t == 0)
    def _():
        zsq_ref[...] = jnp.zeros_like(zsq_ref)

    zsq_ref[...] += jnp.full((1, 1), 1.0) * jnp.sum(z * z)


def _router(x2d, Wr, br):
    n = x2d.shape[0]
    d = x2d.shape[1]
    tile = 512
    grid = n // tile
    return pl.pallas_call(
        _router_body,
        grid=(grid,),
        in_specs=[
            pl.BlockSpec((tile, d), lambda t: (t, 0)),
            pl.BlockSpec((d, E), lambda t: (0, 0)),
            pl.BlockSpec((1, E), lambda t: (0, 0)),
        ],
        out_specs=[
            pl.BlockSpec((tile, TOP_K), lambda t: (t, 0)),
            pl.BlockSpec((tile, TOP_K), lambda t: (t, 0)),
            pl.BlockSpec((1, 1), lambda t: (0, 0)),
        ],
        out_shape=[
            jax.ShapeDtypeStruct((n, TOP_K), jnp.int32),
            jax.ShapeDtypeStruct((n, TOP_K), jnp.float32),
            jax.ShapeDtypeStruct((1, 1), jnp.float32),
        ],
    )(x2d, Wr, br.reshape(1, E))


def _expert_body(eot_ref, xg_ref, win_ref, gain_ref, wout_ref, bout_ref,
                 yg_ref):
    xt = xg_ref[...].astype(jnp.bfloat16)
    h = jnp.dot(xt, win_ref[0], preferred_element_type=jnp.float32)
    d = xt.shape[1]
    x1 = h[:, :d]
    x2 = h[:, d:]
    x1 = 0.5 * x1 * (1.0 + jax.lax.erf(x1 * (2.0 ** -0.5)))
    xm = (x1 * x2 * gain_ref[0]).astype(jnp.bfloat16)
    yg_ref[...] = (jnp.dot(xm, wout_ref[0], preferred_element_type=jnp.float32)
                   + bout_ref[0])


def _grouped_matmul(xg, eot, W_in, gain, W_out, b_out):
    pad_n, d = xg.shape
    nt = pad_n // TILE_M
    grid_spec = pltpu.PrefetchScalarGridSpec(
        num_scalar_prefetch=1,
        grid=(nt,),
        in_specs=[
            pl.BlockSpec((TILE_M, d), lambda t, eot: (t, 0)),
            pl.BlockSpec((1, d, 2 * d), lambda t, eot: (eot[t], 0, 0)),
            pl.BlockSpec((1, 1, d), lambda t, eot: (eot[t], 0, 0)),
            pl.BlockSpec((1, d, d), lambda t, eot: (eot[t], 0, 0)),
            pl.BlockSpec((1, 1, d), lambda t, eot: (eot[t], 0, 0)),
        ],
        out_specs=pl.BlockSpec((TILE_M, d), lambda t, eot: (t, 0)),
    )
    return pl.pallas_call(
        _expert_body,
        grid_spec=grid_spec,
        out_shape=jax.ShapeDtypeStruct((pad_n, d), jnp.float32),
        compiler_params=pltpu.CompilerParams(
            dimension_semantics=("arbitrary",)),
    )(eot, xg, W_in.astype(jnp.bfloat16), gain.reshape(E, 1, d),
      W_out.astype(jnp.bfloat16), b_out.reshape(E, 1, d))


def _combine_body(y0_ref, y1_ref, gates_ref, out_ref):
    g = gates_ref[...]
    out_ref[...] = (g[:, 0:1] * y0_ref[...].astype(jnp.float32)
                    + g[:, 1:2] * y1_ref[...].astype(jnp.float32))


def _combine(ygar, gates, n, d):
    tile = 512
    grid = n // tile
    nblk = n // tile
    return pl.pallas_call(
        _combine_body,
        grid=(grid,),
        in_specs=[
            pl.BlockSpec((tile, d), lambda t: (t, 0)),
            pl.BlockSpec((tile, d), lambda t, nblk=nblk: (nblk + t, 0)),
            pl.BlockSpec((tile, TOP_K), lambda t: (t, 0)),
        ],
        out_specs=pl.BlockSpec((tile, d), lambda t: (t, 0)),
        out_shape=jax.ShapeDtypeStruct((n, d), jnp.float32),
        compiler_params=pltpu.CompilerParams(
            dimension_semantics=("arbitrary",)),
    )(ygar, ygar, gates)


def kernel(x, Wr, br, W_in, gain, W_out, b_out):
    b, t, d = x.shape
    n = b * t
    na = n * TOP_K
    pad_n = na + E * TILE_M
    x2d = x.reshape(n, d)

    top_idx, gates, zsq = _router(x2d, Wr, br)
    z_loss = zsq[0, 0] / n

    ids = top_idx.reshape(-1)
    onehot = (ids[:, None]
              == jnp.arange(E, dtype=jnp.int32)[None, :]).astype(jnp.int32)
    ranks_incl = jnp.cumsum(onehot, axis=0)
    rank = jnp.sum(onehot * (ranks_incl - 1), axis=1)
    counts = ranks_incl[-1]
    aligned = ((counts + TILE_M - 1) // TILE_M) * TILE_M
    off_full = jnp.concatenate(
        [jnp.zeros((1,), jnp.int32), jnp.cumsum(aligned)])
    slot = jnp.sum(onehot * off_full[None, :E], axis=1) + rank
    total = off_full[E]

    nt = pad_n // TILE_M
    tile_base = jnp.arange(nt, dtype=jnp.int32) * TILE_M
    eot = jnp.sum((tile_base[:, None] >= off_full[None, 1:]).astype(jnp.int32),
                  axis=1)
    eot = jnp.minimum(eot, E - 1).astype(jnp.int32)
    last_used = jnp.max(jnp.where(counts > 0, jnp.arange(E, dtype=jnp.int32),
                                  0))
    eot = jnp.where(tile_base < total, eot, last_used)

    tokens = (jnp.arange(na, dtype=jnp.int32) // TOP_K)
    sorted_token = _sc_build_table(slot, tokens, pad_n, n)
    xg = _sc_gather_rows(x2d, sorted_token, chunk=40)

    yg = _grouped_matmul(xg, eot, W_in, gain, W_out, b_out)

    islot_kn = slot.reshape(n, TOP_K).T.reshape(-1)
    ygar = _sc_gather_rows(yg, islot_kn, chunk=32)
    final = _combine(ygar, gates, n, d)

    return final.reshape(b, t, d), z_loss


# speedup vs baseline: 1.0364x; 1.0364x over previous
import functools

import jax
import jax.numpy as jnp
from jax import lax
from jax.experimental import pallas as pl
from jax.experimental.pallas import tpu as pltpu
from jax.experimental.pallas import tpu_sc as plsc

E = 8
TOP_K = 2
TILE_M = 256

NC, NS, LANES = 2, 16, 16
NW = NC * NS
_VMESH = dict(core_axis_name="c", subcore_axis_name="s")


def _sc_compiler_params():
    cp = pltpu.CompilerParams()
    if "needs_layout_passes" in pltpu.CompilerParams.__dataclass_fields__:
        import dataclasses
        cp = dataclasses.replace(cp, needs_layout_passes=False)
    return cp


def _sc_build_table(slot, tokens, pad_n, n_tokens):
    na = slot.shape[0]
    mask = n_tokens - 1

    @functools.partial(
        pl.kernel,
        mesh=plsc.VectorSubcoreMesh(**_VMESH),
        out_type=jax.ShapeDtypeStruct((pad_n,), jnp.int32),
        scratch_types=[
            pltpu.VMEM((na,), jnp.int32),
            pltpu.VMEM((na,), jnp.int32),
            pltpu.VMEM((pad_n,), jnp.int32),
        ],
        compiler_params=_sc_compiler_params(),
    )
    def k(slot_hbm, tok_hbm, out_hbm, slot_v, tok_v, table_v):
        wid = lax.axis_index("s") * NC + lax.axis_index("c")

        @pl.when(wid == 0)
        def _():
            pltpu.sync_copy(slot_hbm, slot_v)
            pltpu.sync_copy(tok_hbm, tok_v)

            @pl.loop(0, pad_n // LANES)
            def _(i):
                base = i * LANES
                table_v[pl.ds(base, LANES)] = (
                    (base + lax.iota(jnp.int32, LANES)) & mask)

            @pl.loop(0, na // LANES)
            def _(i):
                idx = slot_v[pl.ds(i * LANES, LANES)]
                val = tok_v[pl.ds(i * LANES, LANES)]
                plsc.store_scatter(table_v, [idx], val)

            pltpu.sync_copy(table_v, out_hbm)

    return k(slot, tokens)


def _sc_gather_rows(table, idx, chunk=64):
    b = idx.shape[0]
    d = table.shape[1]
    b_per_w = b // NW
    nchunk = b_per_w // chunk
    idx3 = idx.reshape(NW, nchunk, chunk)

    nbuf = min(3, nchunk)

    @functools.partial(
        pl.kernel,
        mesh=plsc.VectorSubcoreMesh(**_VMESH),
        out_type=jax.ShapeDtypeStruct((b, d), table.dtype),
        scratch_types=(
            [pltpu.VMEM((nchunk, chunk), jnp.int32)]
            + [pltpu.VMEM((chunk, d), table.dtype)] * nbuf
            + [pltpu.SemaphoreType.DMA] * (2 * nbuf)
        ),
        compiler_params=_sc_compiler_params(),
    )
    def k(table_hbm, idx_hbm, out_hbm, idx_v, *bufs_and_sems):
        bufs = bufs_and_sems[:nbuf]
        gsem = bufs_and_sems[nbuf:2 * nbuf]
        wsem = bufs_and_sems[2 * nbuf:]
        wid = lax.axis_index("s") * NC + lax.axis_index("c")
        base = wid * b_per_w
        pltpu.sync_copy(idx_hbm.at[wid], idx_v)
        for bb in range(nbuf):
            pltpu.async_copy(table_hbm.at[idx_v.at[bb]], bufs[bb], gsem[bb])

        @pl.loop(0, nchunk)
        def _(c):
            for bb in range(nbuf):
                @pl.when(c % nbuf == bb)
                def _():
                    pltpu.make_async_copy(
                        table_hbm.at[idx_v.at[c]], bufs[bb], gsem[bb]).wait()
                    dst = out_hbm.at[pl.ds(base + c * chunk, chunk)]
                    pltpu.async_copy(bufs[bb], dst, wsem[bb])

                    @pl.when(c + nbuf < nchunk)
                    def _():
                        pltpu.make_async_copy(bufs[bb], dst, wsem[bb]).wait()
                        pltpu.async_copy(
                            table_hbm.at[idx_v.at[c + nbuf]],
                            bufs[bb], gsem[bb])

        for bb in range(nbuf):
            c_last = nchunk - 1 - ((nchunk - 1 - bb) % nbuf)
            pltpu.make_async_copy(
                bufs[bb],
                out_hbm.at[pl.ds(base + c_last * chunk, chunk)],
                wsem[bb]).wait()

    return k(table, idx3)


def _router_body(x_ref, wr_ref, br_ref, idx_ref, gates_ref, zsq_ref):
    t = pl.program_id(0)
    logits = jnp.dot(x_ref[...], wr_ref[...],
                     preferred_element_type=jnp.float32) + br_ref[...]
    rows = logits.shape[0]
    lane = jax.lax.broadcasted_iota(jnp.int32, (rows, E), 1)
    v1 = jnp.max(logits, axis=-1, keepdims=True)
    i1 = jnp.min(jnp.where(logits == v1, lane, E), axis=-1, keepdims=True)
    masked = jnp.where(lane == i1, -jnp.inf, logits)
    v2 = jnp.max(masked, axis=-1, keepdims=True)
    i2 = jnp.min(jnp.where(masked == v2, lane, E), axis=-1, keepdims=True)
    e1 = jnp.exp(v2 - v1)
    denom = 1.0 + e1
    idx_ref[...] = jnp.concatenate([i1, i2], axis=-1)
    gates_ref[...] = jnp.concatenate([1.0 / denom, e1 / denom], axis=-1)
    z = v1 + jnp.log1p(e1)

    @pl.when(t == 0)
    def _():
        zsq_ref[...] = jnp.zeros_like(zsq_ref)

    zsq_ref[...] += jnp.full((1, 1), 1.0) * jnp.sum(z * z)


def _router(x2d, Wr, br):
    n = x2d.shape[0]
    d = x2d.shape[1]
    tile = 512
    grid = n // tile
    return pl.pallas_call(
        _router_body,
        grid=(grid,),
        in_specs=[
            pl.BlockSpec((tile, d), lambda t: (t, 0)),
            pl.BlockSpec((d, E), lambda t: (0, 0)),
            pl.BlockSpec((1, E), lambda t: (0, 0)),
        ],
        out_specs=[
            pl.BlockSpec((tile, TOP_K), lambda t: (t, 0)),
            pl.BlockSpec((tile, TOP_K), lambda t: (t, 0)),
            pl.BlockSpec((1, 1), lambda t: (0, 0)),
        ],
        out_shape=[
            jax.ShapeDtypeStruct((n, TOP_K), jnp.int32),
            jax.ShapeDtypeStruct((n, TOP_K), jnp.float32),
            jax.ShapeDtypeStruct((1, 1), jnp.float32),
        ],
    )(x2d, Wr, br.reshape(1, E))


def _cast_body(win_ref, wout_ref, winb_ref, woutb_ref):
    winb_ref[...] = win_ref[...].astype(jnp.bfloat16)
    woutb_ref[...] = wout_ref[...].astype(jnp.bfloat16)


def _cast_weights(W_in, W_out):
    e, d, d2 = W_in.shape
    return pl.pallas_call(
        _cast_body,
        grid=(e,),
        in_specs=[
            pl.BlockSpec((1, d, d2), lambda t: (t, 0, 0)),
            pl.BlockSpec((1, d, d), lambda t: (t, 0, 0)),
        ],
        out_specs=[
            pl.BlockSpec((1, d, d2), lambda t: (t, 0, 0)),
            pl.BlockSpec((1, d, d), lambda t: (t, 0, 0)),
        ],
        out_shape=[
            jax.ShapeDtypeStruct((e, d, d2), jnp.bfloat16),
            jax.ShapeDtypeStruct((e, d, d), jnp.bfloat16),
        ],
        compiler_params=pltpu.CompilerParams(
            dimension_semantics=("arbitrary",)),
    )(W_in, W_out)


def _expert_body(eot_ref, xg_ref, win_ref, gain_ref, wout_ref, bout_ref,
                 yg_ref):
    xt = xg_ref[...].astype(jnp.bfloat16)
    h = jnp.dot(xt, win_ref[0], preferred_element_type=jnp.float32)
    d = xt.shape[1]
    x1 = h[:, :d]
    x2 = h[:, d:]
    x1 = 0.5 * x1 * (1.0 + jax.lax.erf(x1 * (2.0 ** -0.5)))
    xm = (x1 * x2 * gain_ref[0]).astype(jnp.bfloat16)
    yg_ref[...] = (jnp.dot(xm, wout_ref[0], preferred_element_type=jnp.float32)
                   + bout_ref[0])


def _grouped_matmul(xg, eot, W_in, gain, W_out, b_out):
    pad_n, d = xg.shape
    nt = pad_n // TILE_M
    grid_spec = pltpu.PrefetchScalarGridSpec(
        num_scalar_prefetch=1,
        grid=(nt,),
        in_specs=[
            pl.BlockSpec((TILE_M, d), lambda t, eot: (t, 0)),
            pl.BlockSpec((1, d, 2 * d), lambda t, eot: (eot[t], 0, 0)),
            pl.BlockSpec((1, 1, d), lambda t, eot: (eot[t], 0, 0)),
            pl.BlockSpec((1, d, d), lambda t, eot: (eot[t], 0, 0)),
            pl.BlockSpec((1, 1, d), lambda t, eot: (eot[t], 0, 0)),
        ],
        out_specs=pl.BlockSpec((TILE_M, d), lambda t, eot: (t, 0)),
    )
    W_in_b, W_out_b = _cast_weights(W_in, W_out)
    return pl.pallas_call(
        _expert_body,
        grid_spec=grid_spec,
        out_shape=jax.ShapeDtypeStruct((pad_n, d), jnp.float32),
        compiler_params=pltpu.CompilerParams(
            dimension_semantics=("arbitrary",)),
    )(eot, xg, W_in_b, gain.reshape(E, 1, d), W_out_b,
      b_out.reshape(E, 1, d))


def _combine_body(y0_ref, y1_ref, gates_ref, out_ref):
    g = gates_ref[...]
    out_ref[...] = (g[:, 0:1] * y0_ref[...].astype(jnp.float32)
                    + g[:, 1:2] * y1_ref[...].astype(jnp.float32))


def _combine(ygar, gates, n, d):
    tile = 512
    grid = n // tile
    nblk = n // tile
    return pl.pallas_call(
        _combine_body,
        grid=(grid,),
        in_specs=[
            pl.BlockSpec((tile, d), lambda t: (t, 0)),
            pl.BlockSpec((tile, d), lambda t, nblk=nblk: (nblk + t, 0)),
            pl.BlockSpec((tile, TOP_K), lambda t: (t, 0)),
        ],
        out_specs=pl.BlockSpec((tile, d), lambda t: (t, 0)),
        out_shape=jax.ShapeDtypeStruct((n, d), jnp.float32),
        compiler_params=pltpu.CompilerParams(
            dimension_semantics=("arbitrary",)),
    )(ygar, ygar, gates)


def kernel(x, Wr, br, W_in, gain, W_out, b_out):
    b, t, d = x.shape
    n = b * t
    na = n * TOP_K
    pad_n = na + E * TILE_M
    x2d = x.reshape(n, d)

    top_idx, gates, zsq = _router(x2d, Wr, br)
    z_loss = zsq[0, 0] / n

    ids = top_idx.reshape(-1)
    onehot = (ids[:, None]
              == jnp.arange(E, dtype=jnp.int32)[None, :]).astype(jnp.int32)
    ranks_incl = jnp.cumsum(onehot, axis=0)
    rank = jnp.sum(onehot * (ranks_incl - 1), axis=1)
    counts = ranks_incl[-1]
    aligned = ((counts + TILE_M - 1) // TILE_M) * TILE_M
    off_full = jnp.concatenate(
        [jnp.zeros((1,), jnp.int32), jnp.cumsum(aligned)])
    slot = jnp.sum(onehot * off_full[None, :E], axis=1) + rank
    total = off_full[E]

    nt = pad_n // TILE_M
    tile_base = jnp.arange(nt, dtype=jnp.int32) * TILE_M
    eot = jnp.sum((tile_base[:, None] >= off_full[None, 1:]).astype(jnp.int32),
                  axis=1)
    eot = jnp.minimum(eot, E - 1).astype(jnp.int32)
    last_used = jnp.max(jnp.where(counts > 0, jnp.arange(E, dtype=jnp.int32),
                                  0))
    eot = jnp.where(tile_base < total, eot, last_used)

    tokens = (jnp.arange(na, dtype=jnp.int32) // TOP_K)
    sorted_token = _sc_build_table(slot, tokens, pad_n, n)
    xg = _sc_gather_rows(x2d, sorted_token, chunk=40)

    yg = _grouped_matmul(xg, eot, W_in, gain, W_out, b_out)

    islot_kn = slot.reshape(n, TOP_K).T.reshape(-1)
    ygar = _sc_gather_rows(yg, islot_kn, chunk=32)
    final = _combine(ygar, gates, n, d)

    return final.reshape(b, t, d), z_loss


# speedup vs baseline: 1.0663x; 1.0289x over previous
import functools

import jax
import jax.numpy as jnp
from jax import lax
from jax.experimental import pallas as pl
from jax.experimental.pallas import tpu as pltpu
from jax.experimental.pallas import tpu_sc as plsc

E = 8
TOP_K = 2
TILE_M = 256

NC, NS, LANES = 2, 16, 16
NW = NC * NS
_VMESH = dict(core_axis_name="c", subcore_axis_name="s")


def _sc_compiler_params():
    cp = pltpu.CompilerParams()
    if "needs_layout_passes" in pltpu.CompilerParams.__dataclass_fields__:
        import dataclasses
        cp = dataclasses.replace(cp, needs_layout_passes=False)
    return cp


def _sc_build_table(slot, tokens, pad_n, n_tokens):
    na = slot.shape[0]
    mask = n_tokens - 1

    @functools.partial(
        pl.kernel,
        mesh=plsc.VectorSubcoreMesh(**_VMESH),
        out_type=jax.ShapeDtypeStruct((pad_n,), jnp.int32),
        scratch_types=[
            pltpu.VMEM((na,), jnp.int32),
            pltpu.VMEM((na,), jnp.int32),
            pltpu.VMEM((pad_n,), jnp.int32),
        ],
        compiler_params=_sc_compiler_params(),
    )
    def k(slot_hbm, tok_hbm, out_hbm, slot_v, tok_v, table_v):
        wid = lax.axis_index("s") * NC + lax.axis_index("c")

        @pl.when(wid == 0)
        def _():
            pltpu.sync_copy(slot_hbm, slot_v)
            pltpu.sync_copy(tok_hbm, tok_v)

            @pl.loop(0, pad_n // LANES)
            def _(i):
                base = i * LANES
                table_v[pl.ds(base, LANES)] = (
                    (base + lax.iota(jnp.int32, LANES)) & mask)

            @pl.loop(0, na // LANES)
            def _(i):
                idx = slot_v[pl.ds(i * LANES, LANES)]
                val = tok_v[pl.ds(i * LANES, LANES)]
                plsc.store_scatter(table_v, [idx], val)

            pltpu.sync_copy(table_v, out_hbm)

    return k(slot, tokens)


def _sc_gather_rows(table, idx, chunk=64):
    b = idx.shape[0]
    d = table.shape[1]
    b_per_w = b // NW
    nchunk = b_per_w // chunk
    idx3 = idx.reshape(NW, nchunk, chunk)

    nbuf = min(3, nchunk)

    @functools.partial(
        pl.kernel,
        mesh=plsc.VectorSubcoreMesh(**_VMESH),
        out_type=jax.ShapeDtypeStruct((b, d), table.dtype),
        scratch_types=(
            [pltpu.VMEM((nchunk, chunk), jnp.int32)]
            + [pltpu.VMEM((chunk, d), table.dtype)] * nbuf
            + [pltpu.SemaphoreType.DMA] * (2 * nbuf)
        ),
        compiler_params=_sc_compiler_params(),
    )
    def k(table_hbm, idx_hbm, out_hbm, idx_v, *bufs_and_sems):
        bufs = bufs_and_sems[:nbuf]
        gsem = bufs_and_sems[nbuf:2 * nbuf]
        wsem = bufs_and_sems[2 * nbuf:]
        wid = lax.axis_index("s") * NC + lax.axis_index("c")
        base = wid * b_per_w
        pltpu.sync_copy(idx_hbm.at[wid], idx_v)
        for bb in range(nbuf):
            pltpu.async_copy(table_hbm.at[idx_v.at[bb]], bufs[bb], gsem[bb])

        @pl.loop(0, nchunk)
        def _(c):
            for bb in range(nbuf):
                @pl.when(c % nbuf == bb)
                def _():
                    pltpu.make_async_copy(
                        table_hbm.at[idx_v.at[c]], bufs[bb], gsem[bb]).wait()
                    dst = out_hbm.at[pl.ds(base + c * chunk, chunk)]
                    pltpu.async_copy(bufs[bb], dst, wsem[bb])

                    @pl.when(c + nbuf < nchunk)
                    def _():
                        pltpu.make_async_copy(bufs[bb], dst, wsem[bb]).wait()
                        pltpu.async_copy(
                            table_hbm.at[idx_v.at[c + nbuf]],
                            bufs[bb], gsem[bb])

        for bb in range(nbuf):
            c_last = nchunk - 1 - ((nchunk - 1 - bb) % nbuf)
            pltpu.make_async_copy(
                bufs[bb],
                out_hbm.at[pl.ds(base + c_last * chunk, chunk)],
                wsem[bb]).wait()

    return k(table, idx3)


def _router_body(x_ref, wr_ref, br_ref, idx_ref, gates_ref, zsq_ref):
    t = pl.program_id(0)
    logits = jnp.dot(x_ref[...], wr_ref[...],
                     preferred_element_type=jnp.float32) + br_ref[...]
    rows = logits.shape[0]
    lane = jax.lax.broadcasted_iota(jnp.int32, (rows, E), 1)
    v1 = jnp.max(logits, axis=-1, keepdims=True)
    i1 = jnp.min(jnp.where(logits == v1, lane, E), axis=-1, keepdims=True)
    masked = jnp.where(lane == i1, -jnp.inf, logits)
    v2 = jnp.max(masked, axis=-1, keepdims=True)
    i2 = jnp.min(jnp.where(masked == v2, lane, E), axis=-1, keepdims=True)
    e1 = jnp.exp(v2 - v1)
    denom = 1.0 + e1
    idx_ref[...] = jnp.concatenate([i1, i2], axis=-1)
    gates_ref[...] = jnp.concatenate([1.0 / denom, e1 / denom], axis=-1)
    z = v1 + jnp.log1p(e1)

    @pl.when(t == 0)
    def _():
        zsq_ref[...] = jnp.zeros_like(zsq_ref)

    zsq_ref[...] += jnp.full((1, 1), 1.0) * jnp.sum(z * z)


def _router(x2d, Wr, br):
    n = x2d.shape[0]
    d = x2d.shape[1]
    tile = 512
    grid = n // tile
    return pl.pallas_call(
        _router_body,
        grid=(grid,),
        in_specs=[
            pl.BlockSpec((tile, d), lambda t: (t, 0)),
            pl.BlockSpec((d, E), lambda t: (0, 0)),
            pl.BlockSpec((1, E), lambda t: (0, 0)),
        ],
        out_specs=[
            pl.BlockSpec((tile, TOP_K), lambda t: (t, 0)),
            pl.BlockSpec((tile, TOP_K), lambda t: (t, 0)),
            pl.BlockSpec((1, 1), lambda t: (0, 0)),
        ],
        out_shape=[
            jax.ShapeDtypeStruct((n, TOP_K), jnp.int32),
            jax.ShapeDtypeStruct((n, TOP_K), jnp.float32),
            jax.ShapeDtypeStruct((1, 1), jnp.float32),
        ],
    )(x2d, Wr, br.reshape(1, E))


def _cast_body(win_ref, winb_ref):
    winb_ref[...] = win_ref[...].astype(jnp.bfloat16)


def _cast_weights(W_in):
    e, d, d2 = W_in.shape
    return pl.pallas_call(
        _cast_body,
        grid=(e,),
        in_specs=[pl.BlockSpec((1, d, d2), lambda t: (t, 0, 0))],
        out_specs=pl.BlockSpec((1, d, d2), lambda t: (t, 0, 0)),
        out_shape=jax.ShapeDtypeStruct((e, d, d2), jnp.bfloat16),
        compiler_params=pltpu.CompilerParams(
            dimension_semantics=("arbitrary",)),
    )(W_in)


def _expert_body(eot_ref, xg_ref, win_ref, gain_ref, wout_ref, bout_ref,
                 yg_ref):
    xt = xg_ref[...].astype(jnp.bfloat16)
    h = jnp.dot(xt, win_ref[0], preferred_element_type=jnp.float32)
    d = xt.shape[1]
    x1 = h[:, :d]
    x2 = h[:, d:]
    x1 = 0.5 * x1 * (1.0 + jax.lax.erf(x1 * (2.0 ** -0.5)))
    xm = x1 * x2 * gain_ref[0]
    yg_ref[...] = (jnp.dot(xm, wout_ref[0], preferred_element_type=jnp.float32)
                   + bout_ref[0])


def _grouped_matmul(xg, eot, W_in, gain, W_out, b_out):
    pad_n, d = xg.shape
    nt = pad_n // TILE_M
    grid_spec = pltpu.PrefetchScalarGridSpec(
        num_scalar_prefetch=1,
        grid=(nt,),
        in_specs=[
            pl.BlockSpec((TILE_M, d), lambda t, eot: (t, 0)),
            pl.BlockSpec((1, d, 2 * d), lambda t, eot: (eot[t], 0, 0)),
            pl.BlockSpec((1, 1, d), lambda t, eot: (eot[t], 0, 0)),
            pl.BlockSpec((1, d, d), lambda t, eot: (eot[t], 0, 0)),
            pl.BlockSpec((1, 1, d), lambda t, eot: (eot[t], 0, 0)),
        ],
        out_specs=pl.BlockSpec((TILE_M, d), lambda t, eot: (t, 0)),
    )
    W_in_b = _cast_weights(W_in)
    return pl.pallas_call(
        _expert_body,
        grid_spec=grid_spec,
        out_shape=jax.ShapeDtypeStruct((pad_n, d), jnp.float32),
        compiler_params=pltpu.CompilerParams(
            dimension_semantics=("arbitrary",)),
    )(eot, xg, W_in_b, gain.reshape(E, 1, d), W_out,
      b_out.reshape(E, 1, d))


def _combine_body(y0_ref, y1_ref, gates_ref, out_ref):
    g = gates_ref[...]
    out_ref[...] = (g[:, 0:1] * y0_ref[...].astype(jnp.float32)
                    + g[:, 1:2] * y1_ref[...].astype(jnp.float32))


def _combine(ygar, gates, n, d):
    tile = 512
    grid = n // tile
    nblk = n // tile
    return pl.pallas_call(
        _combine_body,
        grid=(grid,),
        in_specs=[
            pl.BlockSpec((tile, d), lambda t: (t, 0)),
            pl.BlockSpec((tile, d), lambda t, nblk=nblk: (nblk + t, 0)),
            pl.BlockSpec((tile, TOP_K), lambda t: (t, 0)),
        ],
        out_specs=pl.BlockSpec((tile, d), lambda t: (t, 0)),
        out_shape=jax.ShapeDtypeStruct((n, d), jnp.float32),
        compiler_params=pltpu.CompilerParams(
            dimension_semantics=("arbitrary",)),
    )(ygar, ygar, gates)


def kernel(x, Wr, br, W_in, gain, W_out, b_out):
    b, t, d = x.shape
    n = b * t
    na = n * TOP_K
    pad_n = na + E * TILE_M
    x2d = x.reshape(n, d)

    top_idx, gates, zsq = _router(x2d, Wr, br)
    z_loss = zsq[0, 0] / n

    ids = top_idx.T.reshape(-1)
    onehot = (ids[:, None]
              == jnp.arange(E, dtype=jnp.int32)[None, :]).astype(jnp.int32)
    ranks_incl = jnp.cumsum(onehot, axis=0)
    rank = jnp.sum(onehot * (ranks_incl - 1), axis=1)
    counts = ranks_incl[-1]
    aligned = ((counts + TILE_M - 1) // TILE_M) * TILE_M
    off_full = jnp.concatenate(
        [jnp.zeros((1,), jnp.int32), jnp.cumsum(aligned)])
    slot = jnp.sum(onehot * off_full[None, :E], axis=1) + rank
    total = off_full[E]

    nt = pad_n // TILE_M
    tile_base = jnp.arange(nt, dtype=jnp.int32) * TILE_M
    eot = jnp.sum((tile_base[:, None] >= off_full[None, 1:]).astype(jnp.int32),
                  axis=1)
    eot = jnp.minimum(eot, E - 1).astype(jnp.int32)
    last_used = jnp.max(jnp.where(counts > 0, jnp.arange(E, dtype=jnp.int32),
                                  0))
    eot = jnp.where(tile_base < total, eot, last_used)

    tokens = (jnp.arange(na, dtype=jnp.int32) % n)
    sorted_token = _sc_build_table(slot, tokens, pad_n, n)
    xg = _sc_gather_rows(x2d, sorted_token, chunk=40)

    yg = _grouped_matmul(xg, eot, W_in, gain, W_out, b_out)

    ygar = _sc_gather_rows(yg, slot, chunk=32)
    final = _combine(ygar, gates, n, d)

    return final.reshape(b, t, d), z_loss


# speedup vs baseline: 1.0858x; 1.0183x over previous
import functools

import jax
import jax.numpy as jnp
from jax import lax
from jax.experimental import pallas as pl
from jax.experimental.pallas import tpu as pltpu
from jax.experimental.pallas import tpu_sc as plsc

E = 8
TOP_K = 2
TILE_M = 256

NC, NS, LANES = 2, 16, 16
NW = NC * NS
_VMESH = dict(core_axis_name="c", subcore_axis_name="s")


def _sc_compiler_params():
    cp = pltpu.CompilerParams()
    if "needs_layout_passes" in pltpu.CompilerParams.__dataclass_fields__:
        import dataclasses
        cp = dataclasses.replace(cp, needs_layout_passes=False)
    return cp


def _sc_build_table(slot, tokens, pad_n, n_tokens):
    na = slot.shape[0]
    mask = n_tokens - 1

    @functools.partial(
        pl.kernel,
        mesh=plsc.VectorSubcoreMesh(**_VMESH),
        out_type=jax.ShapeDtypeStruct((pad_n,), jnp.int32),
        scratch_types=[
            pltpu.VMEM((na,), jnp.int32),
            pltpu.VMEM((na,), jnp.int32),
            pltpu.VMEM((pad_n,), jnp.int32),
        ],
        compiler_params=_sc_compiler_params(),
    )
    def k(slot_hbm, tok_hbm, out_hbm, slot_v, tok_v, table_v):
        wid = lax.axis_index("s") * NC + lax.axis_index("c")

        @pl.when(wid == 0)
        def _():
            pltpu.sync_copy(slot_hbm, slot_v)
            pltpu.sync_copy(tok_hbm, tok_v)

            @pl.loop(0, pad_n // LANES)
            def _(i):
                base = i * LANES
                table_v[pl.ds(base, LANES)] = (
                    (base + lax.iota(jnp.int32, LANES)) & mask)

            @pl.loop(0, na // LANES)
            def _(i):
                idx = slot_v[pl.ds(i * LANES, LANES)]
                val = tok_v[pl.ds(i * LANES, LANES)]
                plsc.store_scatter(table_v, [idx], val)

            pltpu.sync_copy(table_v, out_hbm)

    return k(slot, tokens)


def _sc_gather_rows(table, idx, chunk=64):
    b = idx.shape[0]
    d = table.shape[1]
    b_per_w = b // NW
    nchunk = b_per_w // chunk
    idx3 = idx.reshape(NW, nchunk, chunk)

    nbuf = min(3, nchunk)

    @functools.partial(
        pl.kernel,
        mesh=plsc.VectorSubcoreMesh(**_VMESH),
        out_type=jax.ShapeDtypeStruct((b, d), table.dtype),
        scratch_types=(
            [pltpu.VMEM((nchunk, chunk), jnp.int32)]
            + [pltpu.VMEM((chunk, d), table.dtype)] * nbuf
            + [pltpu.SemaphoreType.DMA] * (2 * nbuf)
        ),
        compiler_params=_sc_compiler_params(),
    )
    def k(table_hbm, idx_hbm, out_hbm, idx_v, *bufs_and_sems):
        bufs = bufs_and_sems[:nbuf]
        gsem = bufs_and_sems[nbuf:2 * nbuf]
        wsem = bufs_and_sems[2 * nbuf:]
        wid = lax.axis_index("s") * NC + lax.axis_index("c")
        base = wid * b_per_w
        pltpu.sync_copy(idx_hbm.at[wid], idx_v)
        for bb in range(nbuf):
            pltpu.async_copy(table_hbm.at[idx_v.at[bb]], bufs[bb], gsem[bb])

        @pl.loop(0, nchunk)
        def _(c):
            for bb in range(nbuf):
                @pl.when(c % nbuf == bb)
                def _():
                    pltpu.make_async_copy(
                        table_hbm.at[idx_v.at[c]], bufs[bb], gsem[bb]).wait()
                    dst = out_hbm.at[pl.ds(base + c * chunk, chunk)]
                    pltpu.async_copy(bufs[bb], dst, wsem[bb])

                    @pl.when(c + nbuf < nchunk)
                    def _():
                        pltpu.make_async_copy(bufs[bb], dst, wsem[bb]).wait()
                        pltpu.async_copy(
                            table_hbm.at[idx_v.at[c + nbuf]],
                            bufs[bb], gsem[bb])

        for bb in range(nbuf):
            c_last = nchunk - 1 - ((nchunk - 1 - bb) % nbuf)
            pltpu.make_async_copy(
                bufs[bb],
                out_hbm.at[pl.ds(base + c_last * chunk, chunk)],
                wsem[bb]).wait()

    return k(table, idx3)


def _router_body(x_ref, wr_ref, br_ref, idx_ref, gates_ref, zsq_ref):
    t = pl.program_id(0)
    logits = jnp.dot(x_ref[...], wr_ref[...],
                     preferred_element_type=jnp.float32) + br_ref[...]
    rows = logits.shape[0]
    lane = jax.lax.broadcasted_iota(jnp.int32, (rows, E), 1)
    v1 = jnp.max(logits, axis=-1, keepdims=True)
    i1 = jnp.min(jnp.where(logits == v1, lane, E), axis=-1, keepdims=True)
    masked = jnp.where(lane == i1, -jnp.inf, logits)
    v2 = jnp.max(masked, axis=-1, keepdims=True)
    i2 = jnp.min(jnp.where(masked == v2, lane, E), axis=-1, keepdims=True)
    e1 = jnp.exp(v2 - v1)
    denom = 1.0 + e1
    idx_ref[...] = jnp.concatenate([i1, i2], axis=-1)
    gates_ref[...] = jnp.concatenate([1.0 / denom, e1 / denom], axis=-1)
    z = v1 + jnp.log1p(e1)

    @pl.when(t == 0)
    def _():
        zsq_ref[...] = jnp.zeros_like(zsq_ref)

    zsq_ref[...] += jnp.full((1, 1), 1.0) * jnp.sum(z * z)


def _router(x2d, Wr, br):
    n = x2d.shape[0]
    d = x2d.shape[1]
    tile = 512
    grid = n // tile
    return pl.pallas_call(
        _router_body,
        grid=(grid,),
        in_specs=[
            pl.BlockSpec((tile, d), lambda t: (t, 0)),
            pl.BlockSpec((d, E), lambda t: (0, 0)),
            pl.BlockSpec((1, E), lambda t: (0, 0)),
        ],
        out_specs=[
            pl.BlockSpec((tile, TOP_K), lambda t: (t, 0)),
            pl.BlockSpec((tile, TOP_K), lambda t: (t, 0)),
            pl.BlockSpec((1, 1), lambda t: (0, 0)),
        ],
        out_shape=[
            jax.ShapeDtypeStruct((n, TOP_K), jnp.int32),
            jax.ShapeDtypeStruct((n, TOP_K), jnp.float32),
            jax.ShapeDtypeStruct((1, 1), jnp.float32),
        ],
    )(x2d, Wr, br.reshape(1, E))


def _cast_body(win_ref, winb_ref):
    winb_ref[...] = win_ref[...].astype(jnp.bfloat16)


def _cast_weights(W_in):
    e, d, d2 = W_in.shape
    return pl.pallas_call(
        _cast_body,
        grid=(e,),
        in_specs=[pl.BlockSpec((1, d, d2), lambda t: (t, 0, 0))],
        out_specs=pl.BlockSpec((1, d, d2), lambda t: (t, 0, 0)),
        out_shape=jax.ShapeDtypeStruct((e, d, d2), jnp.bfloat16),
        compiler_params=pltpu.CompilerParams(
            dimension_semantics=("arbitrary",)),
    )(W_in)


def _expert_body(eot_ref, xg_ref, win_ref, gain_ref, wout_ref, bout_ref,
                 yg_ref):
    t = pl.program_id(0)

    @pl.when(eot_ref[t] < E)
    def _():
        xt = xg_ref[...].astype(jnp.bfloat16)
        h = jnp.dot(xt, win_ref[0], preferred_element_type=jnp.float32)
        d = xt.shape[1]
        x1 = h[:, :d]
        x2 = h[:, d:]
        x1 = 0.5 * x1 * (1.0 + jax.lax.erf(x1 * (2.0 ** -0.5)))
        xm = x1 * x2 * gain_ref[0]
        yg_ref[...] = (
            jnp.dot(xm, wout_ref[0], preferred_element_type=jnp.float32)
            + bout_ref[0])


def _grouped_matmul(xg, eot, W_in, gain, W_out, b_out):
    pad_n, d = xg.shape
    nt = pad_n // TILE_M
    grid_spec = pltpu.PrefetchScalarGridSpec(
        num_scalar_prefetch=1,
        grid=(nt,),
        in_specs=[
            pl.BlockSpec((TILE_M, d), lambda t, eot: (t, 0)),
            pl.BlockSpec((1, d, 2 * d), lambda t, eot: (eot[t] % E, 0, 0)),
            pl.BlockSpec((1, 1, d), lambda t, eot: (eot[t] % E, 0, 0)),
            pl.BlockSpec((1, d, d), lambda t, eot: (eot[t] % E, 0, 0)),
            pl.BlockSpec((1, 1, d), lambda t, eot: (eot[t] % E, 0, 0)),
        ],
        out_specs=pl.BlockSpec((TILE_M, d), lambda t, eot: (t, 0)),
    )
    W_in_b = _cast_weights(W_in)
    return pl.pallas_call(
        _expert_body,
        grid_spec=grid_spec,
        out_shape=jax.ShapeDtypeStruct((pad_n, d), jnp.float32),
        compiler_params=pltpu.CompilerParams(
            dimension_semantics=("arbitrary",)),
    )(eot, xg, W_in_b, gain.reshape(E, 1, d), W_out,
      b_out.reshape(E, 1, d))


def _combine_body(y0_ref, y1_ref, gates_ref, out_ref):
    g = gates_ref[...]
    out_ref[...] = (g[:, 0:1] * y0_ref[...].astype(jnp.float32)
                    + g[:, 1:2] * y1_ref[...].astype(jnp.float32))


def _combine(ygar, gates, n, d):
    tile = 512
    grid = n // tile
    nblk = n // tile
    return pl.pallas_call(
        _combine_body,
        grid=(grid,),
        in_specs=[
            pl.BlockSpec((tile, d), lambda t: (t, 0)),
            pl.BlockSpec((tile, d), lambda t, nblk=nblk: (nblk + t, 0)),
            pl.BlockSpec((tile, TOP_K), lambda t: (t, 0)),
        ],
        out_specs=pl.BlockSpec((tile, d), lambda t: (t, 0)),
        out_shape=jax.ShapeDtypeStruct((n, d), jnp.float32),
        compiler_params=pltpu.CompilerParams(
            dimension_semantics=("arbitrary",)),
    )(ygar, ygar, gates)


def kernel(x, Wr, br, W_in, gain, W_out, b_out):
    b, t, d = x.shape
    n = b * t
    na = n * TOP_K
    pad_n = na + E * TILE_M
    x2d = x.reshape(n, d)

    top_idx, gates, zsq = _router(x2d, Wr, br)
    z_loss = zsq[0, 0] / n

    ids = top_idx.T.reshape(-1)
    onehot = (ids[:, None]
              == jnp.arange(E, dtype=jnp.int32)[None, :]).astype(jnp.int32)
    ranks_incl = jnp.cumsum(onehot, axis=0)
    rank = jnp.sum(onehot * (ranks_incl - 1), axis=1)
    counts = ranks_incl[-1]
    aligned = ((counts + TILE_M - 1) // TILE_M) * TILE_M
    off_full = jnp.concatenate(
        [jnp.zeros((1,), jnp.int32), jnp.cumsum(aligned)])
    slot = jnp.sum(onehot * off_full[None, :E], axis=1) + rank
    total = off_full[E]

    nt = pad_n // TILE_M
    tile_base = jnp.arange(nt, dtype=jnp.int32) * TILE_M
    eot = jnp.sum((tile_base[:, None] >= off_full[None, 1:]).astype(jnp.int32),
                  axis=1)
    eot = jnp.minimum(eot, E - 1).astype(jnp.int32)
    last_used = jnp.max(jnp.where(counts > 0, jnp.arange(E, dtype=jnp.int32),
                                  0))
    eot = jnp.where(tile_base < total, eot, last_used + E)

    tokens = (jnp.arange(na, dtype=jnp.int32) % n)
    sorted_token = _sc_build_table(slot, tokens, pad_n, n)
    xg = _sc_gather_rows(x2d, sorted_token, chunk=40)

    yg = _grouped_matmul(xg, eot, W_in, gain, W_out, b_out)

    ygar = _sc_gather_rows(yg, slot, chunk=32)
    final = _combine(ygar, gates, n, d)

    return final.reshape(b, t, d), z_loss
